# Initial kernel scaffold; baseline (speedup 1.0000x reference)
import jax
import jax.numpy as jnp
from jax.experimental import pallas as pl


def kernel(x, protos):
    raise NotImplementedError("write your pallas kernel here")


# speedup vs baseline: 1.3894x; 1.3894x over previous
import functools

import jax
import jax.numpy as jnp
from jax.experimental import pallas as pl


def _reduce_kernel(x_ref, o_ref, *, M, C, scale):
    blk = x_ref[0]
    s = jnp.sum(blk.reshape(M, C, blk.shape[-1]), axis=(0, 2))
    o_ref[0, 0, :] = s * scale


def kernel(x, protos):
    N, M, C, T, V = x.shape
    P = protos.shape[0]
    scale = 1.0 / (M * P)
    xr = x.reshape(N, M * C, T * V)
    out = pl.pallas_call(
        functools.partial(_reduce_kernel, M=M, C=C, scale=scale),
        out_shape=jax.ShapeDtypeStruct((N, 1, C), x.dtype),
        grid=(N,),
        in_specs=[pl.BlockSpec((1, M * C, T * V), lambda i: (i, 0, 0))],
        out_specs=pl.BlockSpec((1, 1, C), lambda i: (i, 0, 0)),
    )(xr)
    return out.reshape(N, C)
